# baseline (speedup 1.0000x reference)
import functools

import jax
import jax.numpy as jnp
from jax import lax
from jax.experimental import pallas as pl
from jax.experimental.pallas import tpu as pltpu
from jax.experimental.pallas import tpu_sc as plsc

BATCH = 16384
LANES = 128
ROWS = BATCH // LANES
NC, NS = 2, 16
NW = NC * NS
RPW = ROWS // NW

_MESH = plsc.VectorSubcoreMesh(core_axis_name="c", subcore_axis_name="s")


IPW = BATCH // NW


@functools.partial(
    pl.kernel,
    out_type=jax.ShapeDtypeStruct((BATCH,), jnp.float32),
    mesh=_MESH,
    scratch_types=[
        pltpu.VMEM((IPW,), jnp.int32),
        pltpu.VMEM((IPW,), jnp.float32),
        pltpu.SemaphoreType.DMA,
    ],
)
def _sc_gather(w_hbm, idx_hbm, out_hbm, idx_v, val_v, sem):
    wid = lax.axis_index("s") * NC + lax.axis_index("c")
    base = wid * IPW
    pltpu.sync_copy(idx_hbm.at[pl.ds(base, IPW)], idx_v)
    pltpu.async_copy(w_hbm.at[idx_v], val_v, sem).wait()
    pltpu.sync_copy(val_v, out_hbm.at[pl.ds(base, IPW)])


def kernel(weights, indices):
    return _sc_gather(weights, indices.astype(jnp.int32))


# speedup vs baseline: 1.0328x; 1.0328x over previous
import functools

import jax
import jax.numpy as jnp
from jax import lax
from jax.experimental import pallas as pl
from jax.experimental.pallas import tpu as pltpu
from jax.experimental.pallas import tpu_sc as plsc

BATCH = 16384
LANES = 128
ROWS = BATCH // LANES
NC, NS = 1, 16
NW = NC * NS
RPW = ROWS // NW

_MESH = plsc.VectorSubcoreMesh(core_axis_name="c", subcore_axis_name="s",
                               num_cores=NC)


IPW = BATCH // NW


@functools.partial(
    pl.kernel,
    out_type=jax.ShapeDtypeStruct((BATCH,), jnp.float32),
    mesh=_MESH,
    scratch_types=[
        pltpu.VMEM((IPW,), jnp.int32),
        pltpu.VMEM((IPW,), jnp.float32),
        pltpu.SemaphoreType.DMA,
    ],
)
def _sc_gather(w_hbm, idx_hbm, out_hbm, idx_v, val_v, sem):
    wid = lax.axis_index("s") * NC + lax.axis_index("c")
    base = wid * IPW
    pltpu.sync_copy(idx_hbm.at[pl.ds(base, IPW)], idx_v)
    pltpu.async_copy(w_hbm.at[idx_v], val_v, sem).wait()
    pltpu.sync_copy(val_v, out_hbm.at[pl.ds(base, IPW)])


def kernel(weights, indices):
    return _sc_gather(weights, indices.astype(jnp.int32))


# speedup vs baseline: 1.0365x; 1.0036x over previous
import functools

import jax
import jax.numpy as jnp
from jax import lax
from jax.experimental import pallas as pl
from jax.experimental.pallas import tpu as pltpu
from jax.experimental.pallas import tpu_sc as plsc

BATCH = 16384
LANES = 128
ROWS = BATCH // LANES
NC, NS = 1, 16
NW = NC * NS
RPW = ROWS // NW

_MESH = plsc.VectorSubcoreMesh(core_axis_name="c", subcore_axis_name="s",
                               num_cores=NC)


IPW = BATCH // NW


@functools.partial(
    pl.kernel,
    out_type=jax.ShapeDtypeStruct((BATCH,), jnp.float32),
    mesh=_MESH,
    scratch_types=[
        pltpu.VMEM((IPW,), jnp.int32),
        pltpu.VMEM((IPW,), jnp.float32),
        pltpu.SemaphoreType.DMA,
    ],
)
def _sc_gather(w_hbm, idx_hbm, out_hbm, idx_v, val_v, sem):
    wid = lax.axis_index("s") * NC + lax.axis_index("c")
    base = wid * IPW
    half = IPW // 2
    pltpu.sync_copy(idx_hbm.at[pl.ds(base, IPW)], idx_v)
    c0 = pltpu.async_copy(w_hbm.at[idx_v.at[pl.ds(0, half)]],
                          val_v.at[pl.ds(0, half)], sem)
    c1 = pltpu.async_copy(w_hbm.at[idx_v.at[pl.ds(half, half)]],
                          val_v.at[pl.ds(half, half)], sem)
    c0.wait()
    c1.wait()
    pltpu.sync_copy(val_v, out_hbm.at[pl.ds(base, IPW)])


def kernel(weights, indices):
    return _sc_gather(weights, indices.astype(jnp.int32))


# speedup vs baseline: 1.0499x; 1.0130x over previous
import functools

import jax
import jax.numpy as jnp
from jax import lax
from jax.experimental import pallas as pl
from jax.experimental.pallas import tpu as pltpu
from jax.experimental.pallas import tpu_sc as plsc

BATCH = 16384
LANES = 128
ROWS = BATCH // LANES
NC, NS = 1, 16
NW = NC * NS
RPW = ROWS // NW

_MESH = plsc.VectorSubcoreMesh(core_axis_name="c", subcore_axis_name="s",
                               num_cores=NC)


IPW = BATCH // NW


@functools.partial(
    pl.kernel,
    out_type=jax.ShapeDtypeStruct((BATCH,), jnp.float32),
    mesh=_MESH,
    scratch_types=[
        pltpu.VMEM((IPW,), jnp.int32),
        pltpu.VMEM((IPW,), jnp.float32),
        pltpu.SemaphoreType.DMA,
        pltpu.SemaphoreType.DMA,
        pltpu.SemaphoreType.DMA,
        pltpu.SemaphoreType.DMA,
    ],
)
def _sc_gather(w_hbm, idx_hbm, out_hbm, idx_v, val_v, i0, i1, g0, g1):
    wid = lax.axis_index("s") * NC + lax.axis_index("c")
    base = wid * IPW
    half = IPW // 2
    l0 = pltpu.async_copy(idx_hbm.at[pl.ds(base, half)],
                          idx_v.at[pl.ds(0, half)], i0)
    l1 = pltpu.async_copy(idx_hbm.at[pl.ds(base + half, half)],
                          idx_v.at[pl.ds(half, half)], i1)
    l0.wait()
    c0 = pltpu.async_copy(w_hbm.at[idx_v.at[pl.ds(0, half)]],
                          val_v.at[pl.ds(0, half)], g0)
    l1.wait()
    c1 = pltpu.async_copy(w_hbm.at[idx_v.at[pl.ds(half, half)]],
                          val_v.at[pl.ds(half, half)], g1)
    c0.wait()
    s0 = pltpu.async_copy(val_v.at[pl.ds(0, half)],
                          out_hbm.at[pl.ds(base, half)], i0)
    c1.wait()
    s1 = pltpu.async_copy(val_v.at[pl.ds(half, half)],
                          out_hbm.at[pl.ds(base + half, half)], i1)
    s0.wait()
    s1.wait()


def kernel(weights, indices):
    return _sc_gather(weights, indices.astype(jnp.int32))
